# baseline (speedup 1.0000x reference)
import jax
import jax.numpy as jnp
from jax.experimental import pallas as pl

VOCAB = 1000
B = 1024
S = 50
BB = 256


def _onehot_body(idx_ref, out_ref):
    idx = idx_ref[0, 0, :]
    cols = jax.lax.broadcasted_iota(jnp.int32, (BB, VOCAB), 1)
    out_ref[0, :, :] = (idx[:, None] == cols).astype(jnp.float32)


def kernel(inp):
    idx_t = jnp.transpose(inp, (1, 0)).reshape(S, 1, B)
    grid = (S, B // BB)
    return pl.pallas_call(
        _onehot_body,
        grid=grid,
        in_specs=[pl.BlockSpec((1, 1, BB), lambda s, b: (s, 0, b))],
        out_specs=pl.BlockSpec((1, BB, VOCAB), lambda s, b: (s, b, 0)),
        out_shape=jax.ShapeDtypeStruct((S, B, VOCAB), jnp.float32),
    )(idx_t)
